# Initial kernel scaffold; baseline (speedup 1.0000x reference)
import functools
import jax
import jax.numpy as jnp
from jax import lax
from jax.experimental import pallas as pl
from jax.experimental.pallas import tpu as pltpu
from jax.experimental.pallas import tpu_sc as plsc

N = 10000
NP = 10240
D = 128
E = 320000
KNN = 16
NKEEP = 7500
PP = 7680
WE = 144
NCLS = 10
WC = 16
BLK = 256
DUMP = 7600

_HI = jax.lax.Precision.HIGHEST
BIGF = jnp.float32(1e30)
BIGI = jnp.int32(1 << 30)


def _dense1_body(x_ref, wv_ref, bv_ref, wl_ref, wrp_ref, e1_ref, fl_ref, fre_ref):
    feats = jnp.maximum(jnp.dot(x_ref[...], wv_ref[...], precision=_HI)
                        + bv_ref[...], 0.0)
    fl_ref[...] = jnp.dot(feats, wl_ref[...], precision=_HI)
    fre_ref[...] = jnp.dot(feats, wrp_ref[...], precision=_HI) + e1_ref[...]


def _dense1(xp, W_vig, b_vig, W_l1, Wrp, e1):
    return pl.pallas_call(
        _dense1_body,
        grid=(NP // BLK,),
        in_specs=[
            pl.BlockSpec((BLK, D), lambda i: (i, 0)),
            pl.BlockSpec((D, D), lambda i: (0, 0)),
            pl.BlockSpec((1, D), lambda i: (0, 0)),
            pl.BlockSpec((D, D), lambda i: (0, 0)),
            pl.BlockSpec((D, WE), lambda i: (0, 0)),
            pl.BlockSpec((1, WE), lambda i: (0, 0)),
        ],
        out_specs=[
            pl.BlockSpec((BLK, D), lambda i: (i, 0)),
            pl.BlockSpec((BLK, WE), lambda i: (i, 0)),
        ],
        out_shape=[
            jax.ShapeDtypeStruct((NP, D), jnp.float32),
            jax.ShapeDtypeStruct((NP, WE), jnp.float32),
        ],
    )(xp, W_vig, b_vig, W_l1, Wrp, e1)


def _segsum1_body(fre, srcr, dstr, zz, out, idx_s, idx_d, rows, sem, acc):
    cid = lax.axis_index("c")
    sid = lax.axis_index("s")
    rows_per_tile = NP // 16
    rb = sid * rows_per_tile
    pltpu.sync_copy(zz.at[pl.ds(rb, rows_per_tile)],
                    acc.at[pl.ds(rb, rows_per_tile)])
    plsc.subcore_barrier()
    wid = cid * 16 + sid
    ebase = wid * (E // 32)

    def chunk(c, carry):
        b = pl.multiple_of(ebase + c * 80, 8)
        pltpu.sync_copy(srcr.at[pl.ds(b, 80)], idx_s)
        pltpu.sync_copy(dstr.at[pl.ds(b, 80)], idx_d)
        pltpu.async_copy(fre.at[idx_s], rows, sem).wait()
        pltpu.sync_copy(rows, acc.at[idx_d], add=True)
        return carry

    lax.fori_loop(0, (E // 32) // 80, chunk, 0)
    plsc.subcore_barrier()
    pltpu.sync_copy(acc.at[pl.ds(rb, rows_per_tile)],
                    out.at[pl.ds(cid * NP + rb, rows_per_tile)])


def _segsum1(fre, src, dst, zeros1):
    mesh = plsc.VectorSubcoreMesh(core_axis_name="c", subcore_axis_name="s")
    k = pl.kernel(
        _segsum1_body,
        out_type=jax.ShapeDtypeStruct((2 * NP, WE), jnp.float32),
        mesh=mesh,
        scratch_types=[
            pltpu.VMEM((80,), jnp.int32),
            pltpu.VMEM((80,), jnp.int32),
            pltpu.VMEM((80, WE), jnp.float32),
            pltpu.SemaphoreType.DMA,
            pltpu.VMEM_SHARED((NP, WE), jnp.float32),
        ],
    )
    return k(fre, src, dst, zeros1)


def _combine1_body(fl_ref, agg_ref, b1_ref, p_ref, hg_ref, sc_ref):
    a = agg_ref[...]
    asum = a[0] + a[1]
    aggf = lax.slice(asum, (0, 0), (BLK, D))
    cnt = lax.slice(asum, (0, D), (BLK, D + 1))
    mean = aggf / jnp.maximum(cnt, 1.0)
    h = fl_ref[...] + mean + b1_ref[...]
    pcol = p_ref[...]
    pnorm = jnp.sqrt(jnp.sum(pcol * pcol))
    s = jnp.dot(h, pcol, precision=_HI) / pnorm
    rowi = (lax.broadcasted_iota(jnp.int32, (BLK, 1), 0)
            + pl.program_id(0) * BLK)
    s = jnp.where(rowi < N, s, -BIGF)
    sc_ref[...] = s
    hg_ref[...] = h * jnp.tanh(s)


def _combine1(fl, aggp, b1r, pcol):
    return pl.pallas_call(
        _combine1_body,
        grid=(NP // BLK,),
        in_specs=[
            pl.BlockSpec((BLK, D), lambda i: (i, 0)),
            pl.BlockSpec((2, BLK, WE), lambda i: (0, i, 0)),
            pl.BlockSpec((1, D), lambda i: (0, 0)),
            pl.BlockSpec((D, 1), lambda i: (0, 0)),
        ],
        out_specs=[
            pl.BlockSpec((BLK, D), lambda i: (i, 0)),
            pl.BlockSpec((BLK, 1), lambda i: (i, 0)),
        ],
        out_shape=[
            jax.ShapeDtypeStruct((NP, D), jnp.float32),
            jax.ShapeDtypeStruct((NP, 1), jnp.float32),
        ],
    )(fl, aggp, b1r, pcol)


def _rank_body(scol_ref, srow_ref, out_ref):
    si = scol_ref[...]
    sj = srow_ref[...]
    gt = sj > si
    eq = sj == si
    ji = lax.broadcasted_iota(jnp.int32, (BLK, NP), 1)
    ii = (lax.broadcasted_iota(jnp.int32, (BLK, NP), 0)
          + pl.program_id(0) * BLK)
    cond = gt | (eq & (ji < ii))
    rank = jnp.sum(cond.astype(jnp.int32), axis=1, keepdims=True)
    out_ref[...] = jnp.where(rank < NKEEP, rank, DUMP)


def _rank(scol, srow):
    return pl.pallas_call(
        _rank_body,
        grid=(NP // BLK,),
        in_specs=[
            pl.BlockSpec((BLK, 1), lambda i: (i, 0)),
            pl.BlockSpec((1, NP), lambda i: (0, 0)),
        ],
        out_specs=pl.BlockSpec((BLK, 1), lambda i: (i, 0)),
        out_shape=jax.ShapeDtypeStruct((NP, 1), jnp.int32),
    )(scol, srow)


def _poolscat_body(hg, sidx, out, idx_v, rows, sem):
    cid = lax.axis_index("c")
    sid = lax.axis_index("s")
    wid = cid * 16 + sid
    base = wid * (NP // 32)
    for c in range((NP // 32) // 64):
        b = pl.multiple_of(base + c * 64, 8)
        pltpu.sync_copy(sidx.at[pl.ds(b, 64)], idx_v)
        pltpu.sync_copy(hg.at[pl.ds(b, 64)], rows)
        pltpu.async_copy(rows, out.at[idx_v], sem).wait()


def _poolscat(hg, sidx):
    mesh = plsc.VectorSubcoreMesh(core_axis_name="c", subcore_axis_name="s")
    k = pl.kernel(
        _poolscat_body,
        out_type=jax.ShapeDtypeStruct((PP, D), jnp.float32),
        mesh=mesh,
        scratch_types=[
            pltpu.VMEM((64,), jnp.int32),
            pltpu.VMEM((64, D), jnp.float32),
            pltpu.SemaphoreType.DMA,
        ],
    )
    return k(hg, sidx)


def _knn_body(q_ref, p_ref, wl_ref, wr_ref, bf_ref, idx_ref, plb_ref, pr_ref):
    q = q_ref[...]
    pall = p_ref[...]
    g = lax.dot_general(q, pall, (((1,), (1,)), ((), ())), precision=_HI)
    ones = jnp.ones((1, D), jnp.float32)
    sqj = lax.dot_general(ones, pall * pall, (((1,), (1,)), ((), ())),
                          precision=_HI)
    cur = sqj - 2.0 * g
    jcol = lax.broadcasted_iota(jnp.int32, (BLK, PP), 1)
    cur = jnp.where(jcol >= NKEEP, BIGF, cur)
    cols = []
    for _ in range(KNN):
        m = jnp.min(cur, axis=1, keepdims=True)
        cand = jnp.where(cur == m, jcol, BIGI)
        sel = jnp.min(cand, axis=1, keepdims=True)
        sel = jnp.minimum(sel, PP - 1)
        cols.append(sel)
        cur = jnp.where(jcol == sel, BIGF, cur)
    idx_ref[...] = jnp.concatenate(cols, axis=1)
    plb_ref[...] = jnp.dot(q, wl_ref[...], precision=_HI) + bf_ref[...]
    pr_ref[...] = jnp.dot(q, wr_ref[...], precision=_HI)


def _knn(pooled, Wlp, Wrp2, bfr):
    return pl.pallas_call(
        _knn_body,
        grid=(PP // BLK,),
        in_specs=[
            pl.BlockSpec((BLK, D), lambda i: (i, 0)),
            pl.BlockSpec((PP, D), lambda i: (0, 0)),
            pl.BlockSpec((D, WC), lambda i: (0, 0)),
            pl.BlockSpec((D, WC), lambda i: (0, 0)),
            pl.BlockSpec((1, WC), lambda i: (0, 0)),
        ],
        out_specs=[
            pl.BlockSpec((BLK, KNN), lambda i: (i, 0)),
            pl.BlockSpec((BLK, WC), lambda i: (i, 0)),
            pl.BlockSpec((BLK, WC), lambda i: (i, 0)),
        ],
        out_shape=[
            jax.ShapeDtypeStruct((PP, KNN), jnp.int32),
            jax.ShapeDtypeStruct((PP, WC), jnp.float32),
            jax.ShapeDtypeStruct((PP, WC), jnp.float32),
        ],
    )(pooled, pooled, Wlp, Wrp2, bfr)


def _segsum2_body(pr, srcr, zz, out, idx_s, idx_d, rows, sem, acc):
    cid = lax.axis_index("c")
    sid = lax.axis_index("s")
    rows_per_tile = PP // 16
    rb = sid * rows_per_tile
    pltpu.sync_copy(zz.at[pl.ds(rb, rows_per_tile)],
                    acc.at[pl.ds(rb, rows_per_tile)])
    plsc.subcore_barrier()
    wid = cid * 16 + sid
    ebase = wid * ((PP * KNN) // 32)

    def chunk(c, carry):
        b = pl.multiple_of(ebase + c * 128, 8)
        pltpu.sync_copy(srcr.at[pl.ds(b, 128)], idx_s)
        for t in range(8):
            idx_d[pl.ds(t * 16, 16)] = jnp.right_shift(
                b + t * 16 + lax.iota(jnp.int32, 16), 4)
        pltpu.async_copy(pr.at[idx_s], rows, sem).wait()
        pltpu.sync_copy(rows, acc.at[idx_d], add=True)
        return carry

    lax.fori_loop(0, ((PP * KNN) // 32) // 128, chunk, 0)
    plsc.subcore_barrier()
    pltpu.sync_copy(acc.at[pl.ds(rb, rows_per_tile)],
                    out.at[pl.ds(cid * PP + rb, rows_per_tile)])


def _segsum2(pr, srcflat, zeros2):
    mesh = plsc.VectorSubcoreMesh(core_axis_name="c", subcore_axis_name="s")
    k = pl.kernel(
        _segsum2_body,
        out_type=jax.ShapeDtypeStruct((2 * PP, WC), jnp.float32),
        mesh=mesh,
        scratch_types=[
            pltpu.VMEM((128,), jnp.int32),
            pltpu.VMEM((128,), jnp.int32),
            pltpu.VMEM((128, WC), jnp.float32),
            pltpu.SemaphoreType.DMA,
            pltpu.VMEM_SHARED((PP, WC), jnp.float32),
        ],
    )
    return k(pr, srcflat, zeros2)


def _combine2_body(plb_ref, agg_ref, out_ref):
    a = agg_ref[...]
    out_ref[...] = plb_ref[...] + (a[0] + a[1]) * jnp.float32(1.0 / KNN)


def _combine2(plb, aggf):
    return pl.pallas_call(
        _combine2_body,
        grid=(PP // BLK,),
        in_specs=[
            pl.BlockSpec((BLK, WC), lambda i: (i, 0)),
            pl.BlockSpec((2, BLK, WC), lambda i: (0, i, 0)),
        ],
        out_specs=pl.BlockSpec((BLK, WC), lambda i: (i, 0)),
        out_shape=jax.ShapeDtypeStruct((PP, WC), jnp.float32),
    )(plb, aggf)


def kernel(x, edge_index, W_vig, b_vig, W_l1, W_r1, b1, p, W_lf, W_rf, bf):
    f32 = jnp.float32
    x = x.astype(f32)
    xp = jnp.concatenate([x, jnp.zeros((NP - N, D), f32)], axis=0)
    src = edge_index[0].astype(jnp.int32)
    dst = edge_index[1].astype(jnp.int32)

    Wrp = jnp.zeros((D, WE), f32).at[:, :D].set(W_r1.astype(f32))
    e1 = jnp.zeros((1, WE), f32).at[0, D].set(1.0)
    b1r = b1.astype(f32).reshape(1, D)
    bvr = b_vig.astype(f32).reshape(1, D)
    pcol = p.astype(f32).reshape(D, 1)
    Wlp = jnp.zeros((D, WC), f32).at[:, :NCLS].set(W_lf.astype(f32))
    Wrp2 = jnp.zeros((D, WC), f32).at[:, :NCLS].set(W_rf.astype(f32))
    bfr = jnp.zeros((1, WC), f32).at[0, :NCLS].set(bf.astype(f32))
    zeros1 = jnp.zeros((NP, WE), f32)
    zeros2 = jnp.zeros((PP, WC), f32)

    fl, fre = _dense1(xp, W_vig.astype(f32), bvr, W_l1.astype(f32), Wrp, e1)

    aggp = _segsum1(fre, src, dst, zeros1).reshape(2, NP, WE)

    hg, sc = _combine1(fl, aggp, b1r, pcol)

    scidx = _rank(sc, sc.reshape(1, NP))

    pooled = _poolscat(hg, scidx.reshape(NP))

    idx, plb, pr = _knn(pooled, Wlp, Wrp2, bfr)

    aggf = _segsum2(pr, idx.reshape(PP * KNN), zeros2).reshape(2, PP, WC)

    outp = _combine2(plb, aggf)
    return outp[:NKEEP, :NCLS]


# speedup vs baseline: 1.2721x; 1.2721x over previous
import jax
import jax.numpy as jnp
from jax import lax
from jax.experimental import pallas as pl
from jax.experimental.pallas import tpu as pltpu
from jax.experimental.pallas import tpu_sc as plsc

N = 10000
NP = 10240
D = 128
E = 320000
KNN = 16
NKEEP = 7500
PP = 7680
WE = 144
NCLS = 10
WC = 16
BLK = 256
DUMP = 7600
RPT = NP // 32
BIGF = 1e30
BIGI = 1 << 30
_HI = jax.lax.Precision.HIGHEST

_SIZES = ([126] * 11 + [123] * 4 + [122]) * 2
_BOUNDS = []
_acc = 0
for _s in _SIZES[:-1]:
    _acc += _s * 80
    _BOUNDS.append(_acc)
_BOUNDS.append(E)


def _dense1_body(x_ref, wv_ref, bv_ref, wl_ref, sel_ref, e1_ref, fl_ref, fe_ref):
    feats = jnp.maximum(jnp.dot(x_ref[...], wv_ref[...]) + bv_ref[...], 0.0)
    fl_ref[...] = jnp.dot(feats, wl_ref[...])
    fe_ref[...] = jnp.dot(feats, sel_ref[...], precision=_HI) + e1_ref[...]


def _dense1(xp, W_vig, bvr, W_l1, sel, e1):
    return pl.pallas_call(
        _dense1_body,
        grid=(NP // BLK,),
        in_specs=[
            pl.BlockSpec((BLK, D), lambda i: (i, 0)),
            pl.BlockSpec((D, D), lambda i: (0, 0)),
            pl.BlockSpec((1, D), lambda i: (0, 0)),
            pl.BlockSpec((D, D), lambda i: (0, 0)),
            pl.BlockSpec((D, WE), lambda i: (0, 0)),
            pl.BlockSpec((1, WE), lambda i: (0, 0)),
        ],
        out_specs=[
            pl.BlockSpec((BLK, D), lambda i: (i, 0)),
            pl.BlockSpec((BLK, WE), lambda i: (i, 0)),
        ],
        out_shape=[
            jax.ShapeDtypeStruct((NP, D), jnp.float32),
            jax.ShapeDtypeStruct((NP, WE), jnp.float32),
        ],
    )(xp, W_vig, bvr, W_l1, sel, e1)


def _fold_body(fext, srcr, dstr, out, src_v, dst_v, cpak, gsrc, rows,
               sem, acc):
    cid = lax.axis_index("c")
    sid = lax.axis_index("s")
    wid = cid * 16 + sid
    lo = wid * RPT
    hi = lo + RPT
    z16 = jnp.zeros((16,), jnp.float32)
    iota16 = lax.iota(jnp.int32, 16)

    def zero_acc(i, carry):
        acc[pl.ds(i * 16, 16)] = z16
        return carry
    lax.fori_loop(0, RPT * (WE // 16), zero_acc, 0)

    def drain(F, batch):
        iota = lax.iota(jnp.int32, 16)
        for j in range(5):
            v = cpak[pl.ds(j * 16, 16)]
            gsrc[pl.ds(j * 16, 16)] = jnp.minimum(v & 16383, N - 1)
        pltpu.async_copy(fext.at[gsrc], rows, sem).wait()

        def add_one(k, carry):
            kv = jnp.full((16,), k, jnp.int32)
            pk = plsc.load_gather(cpak, [kv])
            d = lax.shift_right_logical(pk, 14)
            for r in range(WE // 16):
                col = r * 16 + iota
                row = plsc.load_gather(rows, [kv, col])
                plsc.addupdate_scatter(acc, [d * WE + col], row)
            return carry
        lax.fori_loop(0, batch, add_one, 0)
        return F

    def full_drain(F):
        drain(F, 80)
        ts = cpak[pl.ds(80, 16)]
        cpak[pl.ds(0, 16)] = ts
        return F - 80

    def chunk(c, F):
        b = pl.multiple_of(c * 128, 8)
        pltpu.sync_copy(srcr.at[pl.ds(b, 128)], src_v)
        pltpu.sync_copy(dstr.at[pl.ds(b, 128)], dst_v)
        for g in range(8):
            vd = dst_v[pl.ds(g * 16, 16)]
            vs = src_v[pl.ds(g * 16, 16)]
            m = (vd >= lo) & (vd < hi)
            key = iota16 + jnp.where(m, 0, 64)
            pak = vs | jnp.left_shift(vd - lo, 14)
            cpak[pl.ds(F, 16)] = plsc.sort_key_val(key, pak)[1]
            F = F + jnp.max(plsc.all_reduce_population_count(m))
            F = lax.cond(F >= 80, full_drain, lambda f: f, F)
        return F

    F = lax.fori_loop(0, E // 128, chunk, jnp.int32(0))
    lax.cond(F > 0, lambda f: drain(f, f), lambda f: f, F)
    pltpu.sync_copy(acc, out.at[pl.ds(wid * (RPT * WE), RPT * WE)])


def _fold(fext, src, dst):
    mesh = plsc.VectorSubcoreMesh(core_axis_name="c", subcore_axis_name="s",
                                  num_cores=2, num_subcores=16)
    k = pl.kernel(
        _fold_body,
        out_type=jax.ShapeDtypeStruct((NP * WE,), jnp.float32),
        mesh=mesh,
        compiler_params=pltpu.CompilerParams(use_tc_tiling_on_sc=False, needs_layout_passes=False),
        scratch_types=[
            pltpu.VMEM((128,), jnp.int32),
            pltpu.VMEM((128,), jnp.int32),
            pltpu.VMEM((96,), jnp.int32),
            pltpu.VMEM((80,), jnp.int32),
            pltpu.VMEM((80, WE), jnp.float32),
            pltpu.SemaphoreType.DMA,
            pltpu.VMEM((RPT * WE,), jnp.float32),
        ],
    )
    return k(fext, src, dst)


def _bound_body(agg_ref, e128_ref, b_ref, nk_ref, qk_ref):
    cnt = lax.dot_general(e128_ref[...], agg_ref[...],
                          (((1,), (1,)), ((), ())), precision=_HI)
    ci = cnt
    sh = 1
    while sh < NP:
        ci = ci + jnp.concatenate(
            [jnp.zeros((1, sh), jnp.float32), ci[:, :NP - sh]], axis=1)
        sh *= 2
    cexc = ci - cnt
    b = b_ref[...].astype(jnp.float32)
    le = (cexc <= b).astype(jnp.int32)
    nk = jnp.sum(le, axis=1, keepdims=True) - 1
    ji = lax.broadcasted_iota(jnp.int32, (32, NP), 1)
    cnk = jnp.sum(jnp.where(ji == nk, cexc, 0.0), axis=1, keepdims=True)
    qk = b - cnk
    nk_ref[...] = nk
    qk_ref[...] = qk.astype(jnp.int32)


def _bound(agg2d, e128, bcol):
    return pl.pallas_call(
        _bound_body,
        grid=(1,),
        in_specs=[
            pl.BlockSpec((NP, WE), lambda i: (0, 0)),
            pl.BlockSpec((1, WE), lambda i: (0, 0)),
            pl.BlockSpec((32, 1), lambda i: (0, 0)),
        ],
        out_specs=[
            pl.BlockSpec((32, 1), lambda i: (0, 0)),
            pl.BlockSpec((32, 1), lambda i: (0, 0)),
        ],
        out_shape=[
            jax.ShapeDtypeStruct((32, 1), jnp.int32),
            jax.ShapeDtypeStruct((32, 1), jnp.int32),
        ],
    )(agg2d, e128, bcol)


def _fixup_body(fext, srcr, dstr, nkq, out, src_v, dst_v, buf, rows, rowb,
                nq_v, sem):
    cid = lax.axis_index("c")
    sid = lax.axis_index("s")
    t = cid * 16 + sid
    pltpu.sync_copy(nkq, nq_v)
    tv = jnp.full((16,), t, jnp.int32)
    n = plsc.load_gather(nq_v, [tv])
    q = plsc.load_gather(nq_v, [tv + 32])
    zi16 = jnp.zeros((16,), jnp.int32)
    for i in range(128):
        buf[pl.ds(i * 16, 16)] = zi16

    def chunk(c, F):
        b = pl.multiple_of(c * 128, 8)
        pltpu.sync_copy(srcr.at[pl.ds(b, 128)], src_v)
        pltpu.sync_copy(dstr.at[pl.ds(b, 128)], dst_v)

        def take(F):
            iota16 = lax.iota(jnp.int32, 16)
            for g in range(8):
                vd = dst_v[pl.ds(g * 16, 16)]
                vs = src_v[pl.ds(g * 16, 16)]
                m = vd == n
                key = iota16 + jnp.where(m, 0, 64)
                buf[pl.ds(F, 16)] = plsc.sort_key_val(key, vs)[1]
                F = F + jnp.max(plsc.all_reduce_population_count(m))
            return F
        return lax.cond(F < 1900, take, lambda f: f, F)

    F = lax.fori_loop(0, E // 128, chunk, jnp.int32(0))

    nz = [jnp.zeros((16,), jnp.float32)] * (WE // 16)

    def fold_chunk(c, carry):
        accs, stas = carry
        pltpu.async_copy(fext.at[buf.at[pl.ds(c * 80, 80)]], rows, sem).wait()

        def fold_one(i, carry2):
            accs, stas = carry2
            egv = jnp.full((16,), c * 80 + i, jnp.int32)
            isq = egv == q
            valid = egv < F
            iv = jnp.full((16,), i, jnp.int32)
            iota = lax.iota(jnp.int32, 16)
            na, ns = [], []
            for r in range(WE // 16):
                a, s = accs[r], stas[r]
                s = jnp.where(isq, a, s)
                a = jnp.where(isq, 0.0, a)
                row = plsc.load_gather(rows, [iv, r * 16 + iota])
                a = a + jnp.where(valid, row, 0.0)
                na.append(a)
                ns.append(s)
            return (na, ns)
        return lax.fori_loop(0, 80, fold_one, (accs, stas))

    nch = (F + 79) // 80
    accs, stas = lax.fori_loop(0, nch, fold_chunk, (nz, nz))
    for r in range(WE // 16):
        rowb[pl.ds(r * 16, 16)] = stas[r] + accs[r]
    pltpu.sync_copy(rowb, out.at[pl.ds(t * WE, WE)])


def _fixup(fext, src, dst, nkq):
    mesh = plsc.VectorSubcoreMesh(core_axis_name="c", subcore_axis_name="s",
                                  num_cores=2, num_subcores=16)
    k = pl.kernel(
        _fixup_body,
        out_type=jax.ShapeDtypeStruct((32 * WE,), jnp.float32),
        mesh=mesh,
        compiler_params=pltpu.CompilerParams(use_tc_tiling_on_sc=False, needs_layout_passes=False),
        scratch_types=[
            pltpu.VMEM((128,), jnp.int32),
            pltpu.VMEM((128,), jnp.int32),
            pltpu.VMEM((2048,), jnp.int32),
            pltpu.VMEM((80, WE), jnp.float32),
            pltpu.VMEM((WE,), jnp.float32),
            pltpu.VMEM((64,), jnp.int32),
            pltpu.SemaphoreType.DMA,
        ],
    )
    return k(fext, src, dst, nkq)


def _combine1_body(fl_ref, agg_ref, fix_ref, nk_ref, wr_ref, b1_ref, p_ref,
                   hg_ref, sc_ref):
    rowids = (lax.broadcasted_iota(jnp.int32, (BLK, 1), 0)
              + pl.program_id(0) * BLK)
    m_all = (rowids == nk_ref[...]).astype(jnp.float32)
    corr = jnp.dot(m_all, fix_ref[...], precision=_HI)
    anym = jnp.sum(m_all, axis=1, keepdims=True) > 0.0
    a = jnp.where(anym, corr, agg_ref[...])
    aggf = lax.slice(a, (0, 0), (BLK, D))
    cnt = lax.slice(a, (0, D), (BLK, D + 1))
    mean = aggf / jnp.maximum(cnt, 1.0)
    h = fl_ref[...] + jnp.dot(mean, wr_ref[...]) + b1_ref[...]
    pcol = p_ref[...]
    s = jnp.dot(h, pcol) / jnp.sqrt(jnp.sum(pcol * pcol))
    s = jnp.where(rowids < N, s, -BIGF)
    sc_ref[...] = s
    hg_ref[...] = h * jnp.tanh(s)


def _combine1(fl, agg2d, fix, nkrow, W_r1, b1r, pcol):
    return pl.pallas_call(
        _combine1_body,
        grid=(NP // BLK,),
        in_specs=[
            pl.BlockSpec((BLK, D), lambda i: (i, 0)),
            pl.BlockSpec((BLK, WE), lambda i: (i, 0)),
            pl.BlockSpec((32, WE), lambda i: (0, 0)),
            pl.BlockSpec((1, 32), lambda i: (0, 0)),
            pl.BlockSpec((D, D), lambda i: (0, 0)),
            pl.BlockSpec((1, D), lambda i: (0, 0)),
            pl.BlockSpec((D, 1), lambda i: (0, 0)),
        ],
        out_specs=[
            pl.BlockSpec((BLK, D), lambda i: (i, 0)),
            pl.BlockSpec((BLK, 1), lambda i: (i, 0)),
        ],
        out_shape=[
            jax.ShapeDtypeStruct((NP, D), jnp.float32),
            jax.ShapeDtypeStruct((NP, 1), jnp.float32),
        ],
    )(fl, agg2d, fix, nkrow, W_r1, b1r, pcol)


def _rank_body(scol_ref, srow_ref, out_ref):
    si = scol_ref[...]
    sj = srow_ref[...]
    gt = sj > si
    eq = sj == si
    ji = lax.broadcasted_iota(jnp.int32, (BLK, NP), 1)
    ii = (lax.broadcasted_iota(jnp.int32, (BLK, NP), 0)
          + pl.program_id(0) * BLK)
    cond = gt | (eq & (ji < ii))
    rank = jnp.sum(cond.astype(jnp.int32), axis=1, keepdims=True)
    out_ref[...] = jnp.where(rank < NKEEP, rank, DUMP)


def _rank(scol, srow):
    return pl.pallas_call(
        _rank_body,
        grid=(NP // BLK,),
        in_specs=[
            pl.BlockSpec((BLK, 1), lambda i: (i, 0)),
            pl.BlockSpec((1, NP), lambda i: (0, 0)),
        ],
        out_specs=pl.BlockSpec((BLK, 1), lambda i: (i, 0)),
        out_shape=jax.ShapeDtypeStruct((NP, 1), jnp.int32),
    )(scol, srow)


def _poolscat_body(hg, sidx, out, idx_v, rows, sem):
    cid = lax.axis_index("c")
    sid = lax.axis_index("s")
    wid = cid * 16 + sid
    base = wid * (NP // 32)
    for c in range((NP // 32) // 64):
        b = pl.multiple_of(base + c * 64, 8)
        pltpu.sync_copy(sidx.at[pl.ds(b, 64)], idx_v)
        pltpu.sync_copy(hg.at[pl.ds(b, 64)], rows)
        pltpu.async_copy(rows, out.at[idx_v], sem).wait()


def _poolscat(hg, sidx):
    mesh = plsc.VectorSubcoreMesh(core_axis_name="c", subcore_axis_name="s",
                                  num_cores=2, num_subcores=16)
    k = pl.kernel(
        _poolscat_body,
        out_type=jax.ShapeDtypeStruct((PP, D), jnp.float32),
        mesh=mesh,
        compiler_params=pltpu.CompilerParams(use_tc_tiling_on_sc=False, needs_layout_passes=False),
        scratch_types=[
            pltpu.VMEM((64,), jnp.int32),
            pltpu.VMEM((64, D), jnp.float32),
            pltpu.SemaphoreType.DMA,
        ],
    )
    return k(hg, sidx)


def _knn_body(q_ref, p_ref, idx_ref):
    q = q_ref[...]
    pall = p_ref[...]
    g = lax.dot_general(q, pall, (((1,), (1,)), ((), ())))
    ones = jnp.ones((1, D), jnp.float32)
    sq_row = lax.dot_general(ones, pall * pall, (((1,), (1,)), ((), ())),
                             precision=_HI)
    sq_col = jnp.sum(q * q, axis=1, keepdims=True)
    cur = (sq_col + sq_row) - 2.0 * g
    jcol = lax.broadcasted_iota(jnp.int32, (BLK, PP), 1)
    cur = jnp.where(jcol >= NKEEP, BIGF, cur)
    cols = []
    for _ in range(KNN):
        m = jnp.min(cur, axis=1, keepdims=True)
        cand = jnp.where(cur == m, jcol, BIGI)
        sel = jnp.min(cand, axis=1, keepdims=True)
        sel = jnp.minimum(sel, PP - 1)
        cols.append(sel)
        cur = jnp.where(jcol == sel, BIGF, cur)
    idx_ref[...] = jnp.concatenate(cols, axis=1)


def _knn(pooled):
    return pl.pallas_call(
        _knn_body,
        grid=(PP // BLK,),
        in_specs=[
            pl.BlockSpec((BLK, D), lambda i: (i, 0)),
            pl.BlockSpec((PP, D), lambda i: (0, 0)),
        ],
        out_specs=pl.BlockSpec((BLK, KNN), lambda i: (i, 0)),
        out_shape=jax.ShapeDtypeStruct((PP, KNN), jnp.int32),
    )(pooled, pooled)


def _segsum2_body(pr, srcr, zz, out, idx_s, idx_d, rows, sem, acc):
    cid = lax.axis_index("c")
    sid = lax.axis_index("s")
    rows_per_tile = PP // 16
    rb = sid * rows_per_tile
    pltpu.sync_copy(zz.at[pl.ds(rb, rows_per_tile)],
                    acc.at[pl.ds(rb, rows_per_tile)])
    plsc.subcore_barrier()
    wid = cid * 16 + sid
    ebase = wid * ((PP * KNN) // 32)

    def chunk(c, carry):
        b = pl.multiple_of(ebase + c * 128, 8)
        pltpu.sync_copy(srcr.at[pl.ds(b, 128)], idx_s)
        for t in range(8):
            idx_d[pl.ds(t * 16, 16)] = jnp.right_shift(
                b + t * 16 + lax.iota(jnp.int32, 16), 4)
        pltpu.async_copy(pr.at[idx_s], rows, sem).wait()
        pltpu.sync_copy(rows, acc.at[idx_d], add=True)
        return carry

    lax.fori_loop(0, ((PP * KNN) // 32) // 128, chunk, 0)
    plsc.subcore_barrier()
    pltpu.sync_copy(acc.at[pl.ds(rb, rows_per_tile)],
                    out.at[pl.ds(cid * PP + rb, rows_per_tile)])


def _segsum2(pooled, srcflat, zeros2):
    mesh = plsc.VectorSubcoreMesh(core_axis_name="c", subcore_axis_name="s",
                                  num_cores=2, num_subcores=16)
    k = pl.kernel(
        _segsum2_body,
        out_type=jax.ShapeDtypeStruct((2 * PP, D), jnp.float32),
        mesh=mesh,
        compiler_params=pltpu.CompilerParams(use_tc_tiling_on_sc=False, needs_layout_passes=False),
        scratch_types=[
            pltpu.VMEM((128,), jnp.int32),
            pltpu.VMEM((128,), jnp.int32),
            pltpu.VMEM((128, D), jnp.float32),
            pltpu.SemaphoreType.DMA,
            pltpu.VMEM_SHARED((PP, D), jnp.float32),
        ],
    )
    return k(pooled, srcflat, zeros2)


def _combine2_body(pool_ref, agg_ref, wl_ref, wr_ref, bf_ref, out_ref):
    a = agg_ref[...]
    mean = (a[0] + a[1]) * (1.0 / KNN)
    out_ref[...] = (jnp.dot(pool_ref[...], wl_ref[...])
                    + jnp.dot(mean, wr_ref[...]) + bf_ref[...])


def _combine2(pooled, aggf, Wlp, Wrp2, bfr):
    return pl.pallas_call(
        _combine2_body,
        grid=(PP // BLK,),
        in_specs=[
            pl.BlockSpec((BLK, D), lambda i: (i, 0)),
            pl.BlockSpec((2, BLK, D), lambda i: (0, i, 0)),
            pl.BlockSpec((D, WC), lambda i: (0, 0)),
            pl.BlockSpec((D, WC), lambda i: (0, 0)),
            pl.BlockSpec((1, WC), lambda i: (0, 0)),
        ],
        out_specs=pl.BlockSpec((BLK, WC), lambda i: (i, 0)),
        out_shape=jax.ShapeDtypeStruct((PP, WC), jnp.float32),
    )(pooled, aggf, Wlp, Wrp2, bfr)


def kernel(x, edge_index, W_vig, b_vig, W_l1, W_r1, b1, p, W_lf, W_rf, bf):
    f32 = jnp.float32
    x = x.astype(f32)
    xp = jnp.concatenate([x, jnp.zeros((NP - N, D), f32)], axis=0)
    src = edge_index[0].astype(jnp.int32)
    dst = edge_index[1].astype(jnp.int32)

    sel = jnp.zeros((D, WE), f32).at[:D, :D].set(jnp.eye(D, dtype=f32))
    e1 = jnp.zeros((1, WE), f32).at[0, D].set(1.0)
    b1r = b1.astype(f32).reshape(1, D)
    bvr = b_vig.astype(f32).reshape(1, D)
    pcol = p.astype(f32).reshape(D, 1)
    e128 = jnp.zeros((1, WE), f32).at[0, D].set(1.0)
    bcol = jnp.asarray(_BOUNDS, jnp.int32).reshape(32, 1)
    Wlp = jnp.zeros((D, WC), f32).at[:, :NCLS].set(W_lf.astype(f32))
    Wrp2 = jnp.zeros((D, WC), f32).at[:, :NCLS].set(W_rf.astype(f32))
    bfr = jnp.zeros((1, WC), f32).at[0, :NCLS].set(bf.astype(f32))
    zeros2 = jnp.zeros((PP, D), f32)

    fl, fext = _dense1(xp, W_vig.astype(f32), bvr, W_l1.astype(f32), sel, e1)

    agg2d = _fold(fext, src, dst).reshape(NP, WE)

    nk, qk = _bound(agg2d, e128, bcol)
    nkq = jnp.concatenate([nk.reshape(32), qk.reshape(32)])
    fix = _fixup(fext, src, dst, nkq).reshape(32, WE)

    hg, sc = _combine1(fl, agg2d, fix, nk.reshape(1, 32),
                       W_r1.astype(f32), b1r, pcol)

    scidx = _rank(sc, sc.reshape(1, NP))
    pooled = _poolscat(hg, scidx.reshape(NP))

    idx = _knn(pooled)

    aggf = _segsum2(pooled, idx.reshape(PP * KNN), zeros2).reshape(2, PP, D)
    outp = _combine2(pooled, aggf, Wlp, Wrp2, bfr)
    return outp[:NKEEP, :NCLS]


# speedup vs baseline: 1.4642x; 1.1509x over previous
import jax
import jax.numpy as jnp
from jax import lax
from jax.experimental import pallas as pl
from jax.experimental.pallas import tpu as pltpu
from jax.experimental.pallas import tpu_sc as plsc

N = 10000
NP = 10240
D = 128
E = 320000
KNN = 16
NKEEP = 7500
PP = 7680
WE = 144
NCLS = 10
WC = 16
BLK = 256
DUMP = 7600
RPT = NP // 32
BIGF = 1e30
BIGI = 1 << 30
_HI = jax.lax.Precision.HIGHEST

_SIZES = ([126] * 11 + [123] * 4 + [122]) * 2
_BOUNDS = []
_acc = 0
for _s in _SIZES[:-1]:
    _acc += _s * 80
    _BOUNDS.append(_acc)
_BOUNDS.append(E)


def _dense1_body(x_ref, wv_ref, bv_ref, wl_ref, sel_ref, e1_ref, fl_ref, fe_ref):
    feats = jnp.maximum(jnp.dot(x_ref[...], wv_ref[...]) + bv_ref[...], 0.0)
    fl_ref[...] = jnp.dot(feats, wl_ref[...])
    fe_ref[...] = jnp.dot(feats, sel_ref[...], precision=_HI) + e1_ref[...]


def _dense1(xp, W_vig, bvr, W_l1, sel, e1):
    return pl.pallas_call(
        _dense1_body,
        grid=(NP // BLK,),
        in_specs=[
            pl.BlockSpec((BLK, D), lambda i: (i, 0)),
            pl.BlockSpec((D, D), lambda i: (0, 0)),
            pl.BlockSpec((1, D), lambda i: (0, 0)),
            pl.BlockSpec((D, D), lambda i: (0, 0)),
            pl.BlockSpec((D, WE), lambda i: (0, 0)),
            pl.BlockSpec((1, WE), lambda i: (0, 0)),
        ],
        out_specs=[
            pl.BlockSpec((BLK, D), lambda i: (i, 0)),
            pl.BlockSpec((BLK, WE), lambda i: (i, 0)),
        ],
        out_shape=[
            jax.ShapeDtypeStruct((NP, D), jnp.float32),
            jax.ShapeDtypeStruct((NP, WE), jnp.float32),
        ],
    )(xp, W_vig, bvr, W_l1, sel, e1)


def _fold_body(fext, srcr, dstr, out, src_v, dst_v, cpak, gsrc, rows,
               sem, acc):
    cid = lax.axis_index("c")
    sid = lax.axis_index("s")
    wid = cid * 16 + sid
    lo = wid * RPT
    hi = lo + RPT
    z16 = jnp.zeros((16,), jnp.float32)
    iota16 = lax.iota(jnp.int32, 16)

    def zero_acc(i, carry):
        acc[pl.ds(i * 16, 16)] = z16
        return carry
    lax.fori_loop(0, RPT * (WE // 16), zero_acc, 0)

    def drain(F, batch):
        iota = lax.iota(jnp.int32, 16)
        for j in range(5):
            v = cpak[pl.ds(j * 16, 16)]
            gsrc[pl.ds(j * 16, 16)] = jnp.minimum(v & 16383, N - 1)
        pltpu.async_copy(fext.at[gsrc], rows, sem).wait()

        def add_one(k, carry):
            kv = jnp.full((16,), k, jnp.int32)
            pk = plsc.load_gather(cpak, [kv])
            d = lax.shift_right_logical(pk, 14)
            for r in range(WE // 16):
                col = r * 16 + iota
                row = plsc.load_gather(rows, [kv, col])
                plsc.addupdate_scatter(acc, [d * WE + col], row)
            return carry
        lax.fori_loop(0, batch, add_one, 0)
        return F

    def full_drain(F):
        drain(F, 80)
        ts = cpak[pl.ds(80, 16)]
        cpak[pl.ds(0, 16)] = ts
        return F - 80

    def chunk(c, F):
        b = pl.multiple_of(c * 256, 8)
        pltpu.sync_copy(srcr.at[pl.ds(b, 256)], src_v)
        pltpu.sync_copy(dstr.at[pl.ds(b, 256)], dst_v)
        for g in range(16):
            vd = dst_v[pl.ds(g * 16, 16)]
            vs = src_v[pl.ds(g * 16, 16)]
            m = (vd >= lo) & (vd < hi)
            key = iota16 + jnp.where(m, 0, 64)
            pak = vs | jnp.left_shift(vd - lo, 14)
            cpak[pl.ds(F, 16)] = plsc.sort_key_val(key, pak)[1]
            F = F + jnp.max(plsc.all_reduce_population_count(m))
            F = lax.cond(F >= 80, full_drain, lambda f: f, F)
        return F

    F = lax.fori_loop(0, E // 256, chunk, jnp.int32(0))
    lax.cond(F > 0, lambda f: drain(f, f), lambda f: f, F)
    pltpu.sync_copy(acc, out.at[pl.ds(wid * (RPT * WE), RPT * WE)])


def _fold(fext, src, dst):
    mesh = plsc.VectorSubcoreMesh(core_axis_name="c", subcore_axis_name="s",
                                  num_cores=2, num_subcores=16)
    k = pl.kernel(
        _fold_body,
        out_type=jax.ShapeDtypeStruct((NP * WE,), jnp.float32),
        mesh=mesh,
        compiler_params=pltpu.CompilerParams(use_tc_tiling_on_sc=False, needs_layout_passes=False),
        scratch_types=[
            pltpu.VMEM((256,), jnp.int32),
            pltpu.VMEM((256,), jnp.int32),
            pltpu.VMEM((96,), jnp.int32),
            pltpu.VMEM((80,), jnp.int32),
            pltpu.VMEM((80, WE), jnp.float32),
            pltpu.SemaphoreType.DMA,
            pltpu.VMEM((RPT * WE,), jnp.float32),
        ],
    )
    return k(fext, src, dst)


def _bound_body(agg_ref, e128_ref, b_ref, nk_ref, qk_ref):
    cnt = lax.dot_general(e128_ref[...], agg_ref[...],
                          (((1,), (1,)), ((), ())), precision=_HI)
    ci = cnt
    sh = 1
    while sh < NP:
        ci = ci + jnp.concatenate(
            [jnp.zeros((1, sh), jnp.float32), ci[:, :NP - sh]], axis=1)
        sh *= 2
    cexc = ci - cnt
    b = b_ref[...].astype(jnp.float32)
    le = (cexc <= b).astype(jnp.int32)
    nk = jnp.sum(le, axis=1, keepdims=True) - 1
    ji = lax.broadcasted_iota(jnp.int32, (32, NP), 1)
    cnk = jnp.sum(jnp.where(ji == nk, cexc, 0.0), axis=1, keepdims=True)
    qk = b - cnk
    nk_ref[...] = nk
    qk_ref[...] = qk.astype(jnp.int32)


def _bound(agg2d, e128, bcol):
    return pl.pallas_call(
        _bound_body,
        grid=(1,),
        in_specs=[
            pl.BlockSpec((NP, WE), lambda i: (0, 0)),
            pl.BlockSpec((1, WE), lambda i: (0, 0)),
            pl.BlockSpec((32, 1), lambda i: (0, 0)),
        ],
        out_specs=[
            pl.BlockSpec((32, 1), lambda i: (0, 0)),
            pl.BlockSpec((32, 1), lambda i: (0, 0)),
        ],
        out_shape=[
            jax.ShapeDtypeStruct((32, 1), jnp.int32),
            jax.ShapeDtypeStruct((32, 1), jnp.int32),
        ],
    )(agg2d, e128, bcol)


def _fixup_body(fext, srcr, dstr, nkq, out, src_v, dst_v, buf, rows, rowb,
                nq_v, sem):
    cid = lax.axis_index("c")
    sid = lax.axis_index("s")
    t = cid * 16 + sid
    pltpu.sync_copy(nkq, nq_v)
    tv = jnp.full((16,), t, jnp.int32)
    n = plsc.load_gather(nq_v, [tv])
    q = plsc.load_gather(nq_v, [tv + 32])
    zi16 = jnp.zeros((16,), jnp.int32)
    for i in range(128):
        buf[pl.ds(i * 16, 16)] = zi16

    def chunk(c, F):
        b = pl.multiple_of(c * 256, 8)
        pltpu.sync_copy(dstr.at[pl.ds(b, 256)], dst_v)
        nmatch = jnp.int32(0)
        for g in range(16):
            vd = dst_v[pl.ds(g * 16, 16)]
            nmatch = nmatch + jnp.max(
                plsc.all_reduce_population_count(vd == n))

        def take(F):
            pltpu.sync_copy(srcr.at[pl.ds(b, 256)], src_v)
            iota16 = lax.iota(jnp.int32, 16)
            for g in range(16):
                vd = dst_v[pl.ds(g * 16, 16)]
                vs = src_v[pl.ds(g * 16, 16)]
                m = vd == n
                key = iota16 + jnp.where(m, 0, 64)
                buf[pl.ds(F, 16)] = plsc.sort_key_val(key, vs)[1]
                F = F + jnp.max(plsc.all_reduce_population_count(m))
            return F
        return lax.cond((nmatch > 0) & (F < 1700), take, lambda f: f, F)

    F = lax.fori_loop(0, E // 256, chunk, jnp.int32(0))

    nz = [jnp.zeros((16,), jnp.float32)] * (WE // 16)

    def fold_chunk(c, carry):
        accs, stas = carry
        pltpu.async_copy(fext.at[buf.at[pl.ds(c * 80, 80)]], rows, sem).wait()

        def fold_one(i, carry2):
            accs, stas = carry2
            egv = jnp.full((16,), c * 80 + i, jnp.int32)
            isq = egv == q
            valid = egv < F
            iv = jnp.full((16,), i, jnp.int32)
            iota = lax.iota(jnp.int32, 16)
            na, ns = [], []
            for r in range(WE // 16):
                a, s = accs[r], stas[r]
                s = jnp.where(isq, a, s)
                a = jnp.where(isq, 0.0, a)
                row = plsc.load_gather(rows, [iv, r * 16 + iota])
                a = a + jnp.where(valid, row, 0.0)
                na.append(a)
                ns.append(s)
            return (na, ns)
        return lax.fori_loop(0, 80, fold_one, (accs, stas))

    nch = (F + 79) // 80
    accs, stas = lax.fori_loop(0, nch, fold_chunk, (nz, nz))
    for r in range(WE // 16):
        rowb[pl.ds(r * 16, 16)] = stas[r] + accs[r]
    pltpu.sync_copy(rowb, out.at[pl.ds(t * WE, WE)])


def _fixup(fext, src, dst, nkq):
    mesh = plsc.VectorSubcoreMesh(core_axis_name="c", subcore_axis_name="s",
                                  num_cores=2, num_subcores=16)
    k = pl.kernel(
        _fixup_body,
        out_type=jax.ShapeDtypeStruct((32 * WE,), jnp.float32),
        mesh=mesh,
        compiler_params=pltpu.CompilerParams(use_tc_tiling_on_sc=False, needs_layout_passes=False),
        scratch_types=[
            pltpu.VMEM((256,), jnp.int32),
            pltpu.VMEM((256,), jnp.int32),
            pltpu.VMEM((2048,), jnp.int32),
            pltpu.VMEM((80, WE), jnp.float32),
            pltpu.VMEM((WE,), jnp.float32),
            pltpu.VMEM((64,), jnp.int32),
            pltpu.SemaphoreType.DMA,
        ],
    )
    return k(fext, src, dst, nkq)


def _combine1_body(fl_ref, agg_ref, fix_ref, nk_ref, wr_ref, b1_ref, p_ref,
                   hg_ref, sc_ref):
    rowids = (lax.broadcasted_iota(jnp.int32, (BLK, 1), 0)
              + pl.program_id(0) * BLK)
    m_all = (rowids == nk_ref[...]).astype(jnp.float32)
    corr = jnp.dot(m_all, fix_ref[...], precision=_HI)
    anym = jnp.sum(m_all, axis=1, keepdims=True) > 0.0
    a = jnp.where(anym, corr, agg_ref[...])
    aggf = lax.slice(a, (0, 0), (BLK, D))
    cnt = lax.slice(a, (0, D), (BLK, D + 1))
    mean = aggf / jnp.maximum(cnt, 1.0)
    h = fl_ref[...] + jnp.dot(mean, wr_ref[...]) + b1_ref[...]
    pcol = p_ref[...]
    s = jnp.dot(h, pcol) / jnp.sqrt(jnp.sum(pcol * pcol))
    s = jnp.where(rowids < N, s, -BIGF)
    sc_ref[...] = s
    hg_ref[...] = h * jnp.tanh(s)


def _combine1(fl, agg2d, fix, nkrow, W_r1, b1r, pcol):
    return pl.pallas_call(
        _combine1_body,
        grid=(NP // BLK,),
        in_specs=[
            pl.BlockSpec((BLK, D), lambda i: (i, 0)),
            pl.BlockSpec((BLK, WE), lambda i: (i, 0)),
            pl.BlockSpec((32, WE), lambda i: (0, 0)),
            pl.BlockSpec((1, 32), lambda i: (0, 0)),
            pl.BlockSpec((D, D), lambda i: (0, 0)),
            pl.BlockSpec((1, D), lambda i: (0, 0)),
            pl.BlockSpec((D, 1), lambda i: (0, 0)),
        ],
        out_specs=[
            pl.BlockSpec((BLK, D), lambda i: (i, 0)),
            pl.BlockSpec((BLK, 1), lambda i: (i, 0)),
        ],
        out_shape=[
            jax.ShapeDtypeStruct((NP, D), jnp.float32),
            jax.ShapeDtypeStruct((NP, 1), jnp.float32),
        ],
    )(fl, agg2d, fix, nkrow, W_r1, b1r, pcol)


def _rank_body(scol_ref, srow_ref, out_ref):
    si = scol_ref[...]
    sj = srow_ref[...]
    gt = sj > si
    eq = sj == si
    ji = lax.broadcasted_iota(jnp.int32, (BLK, NP), 1)
    ii = (lax.broadcasted_iota(jnp.int32, (BLK, NP), 0)
          + pl.program_id(0) * BLK)
    cond = gt | (eq & (ji < ii))
    rank = jnp.sum(cond.astype(jnp.int32), axis=1, keepdims=True)
    out_ref[...] = jnp.where(rank < NKEEP, rank, DUMP)


def _rank(scol, srow):
    return pl.pallas_call(
        _rank_body,
        grid=(NP // BLK,),
        in_specs=[
            pl.BlockSpec((BLK, 1), lambda i: (i, 0)),
            pl.BlockSpec((1, NP), lambda i: (0, 0)),
        ],
        out_specs=pl.BlockSpec((BLK, 1), lambda i: (i, 0)),
        out_shape=jax.ShapeDtypeStruct((NP, 1), jnp.int32),
    )(scol, srow)


def _poolscat_body(hg, sidx, out, idx_v, rows, sem):
    cid = lax.axis_index("c")
    sid = lax.axis_index("s")
    wid = cid * 16 + sid
    base = wid * (NP // 32)
    for c in range((NP // 32) // 64):
        b = pl.multiple_of(base + c * 64, 8)
        pltpu.sync_copy(sidx.at[pl.ds(b, 64)], idx_v)
        pltpu.sync_copy(hg.at[pl.ds(b, 64)], rows)
        pltpu.async_copy(rows, out.at[idx_v], sem).wait()


def _poolscat(hg, sidx):
    mesh = plsc.VectorSubcoreMesh(core_axis_name="c", subcore_axis_name="s",
                                  num_cores=2, num_subcores=16)
    k = pl.kernel(
        _poolscat_body,
        out_type=jax.ShapeDtypeStruct((PP, D), jnp.float32),
        mesh=mesh,
        compiler_params=pltpu.CompilerParams(use_tc_tiling_on_sc=False, needs_layout_passes=False),
        scratch_types=[
            pltpu.VMEM((64,), jnp.int32),
            pltpu.VMEM((64, D), jnp.float32),
            pltpu.SemaphoreType.DMA,
        ],
    )
    return k(hg, sidx)


def _knn_body(q_ref, p_ref, idx_ref):
    q = q_ref[...]
    pall = p_ref[...]
    g = lax.dot_general(q, pall, (((1,), (1,)), ((), ())))
    ones = jnp.ones((1, D), jnp.float32)
    sq_row = lax.dot_general(ones, pall * pall, (((1,), (1,)), ((), ())),
                             precision=_HI)
    sq_col = jnp.sum(q * q, axis=1, keepdims=True)
    cur = (sq_col + sq_row) - 2.0 * g
    jcol = lax.broadcasted_iota(jnp.int32, (BLK, PP), 1)
    cur = jnp.where(jcol >= NKEEP, BIGF, cur)
    cols = []
    for _ in range(KNN):
        m = jnp.min(cur, axis=1, keepdims=True)
        cand = jnp.where(cur == m, jcol, BIGI)
        sel = jnp.min(cand, axis=1, keepdims=True)
        sel = jnp.minimum(sel, PP - 1)
        cols.append(sel)
        cur = jnp.where(jcol == sel, BIGF, cur)
    idx_ref[...] = jnp.concatenate(cols, axis=1)


def _knn(pooled):
    return pl.pallas_call(
        _knn_body,
        grid=(PP // BLK,),
        in_specs=[
            pl.BlockSpec((BLK, D), lambda i: (i, 0)),
            pl.BlockSpec((PP, D), lambda i: (0, 0)),
        ],
        out_specs=pl.BlockSpec((BLK, KNN), lambda i: (i, 0)),
        out_shape=jax.ShapeDtypeStruct((PP, KNN), jnp.int32),
    )(pooled, pooled)


def _segsum2_body(pr, srcr, zz, out, idx_s, idx_d, rows, sem, acc):
    cid = lax.axis_index("c")
    sid = lax.axis_index("s")
    rows_per_tile = PP // 16
    rb = sid * rows_per_tile
    pltpu.sync_copy(zz.at[pl.ds(rb, rows_per_tile)],
                    acc.at[pl.ds(rb, rows_per_tile)])
    plsc.subcore_barrier()
    wid = cid * 16 + sid
    ebase = wid * ((PP * KNN) // 32)

    def chunk(c, carry):
        b = pl.multiple_of(ebase + c * 128, 8)
        pltpu.sync_copy(srcr.at[pl.ds(b, 128)], idx_s)
        for t in range(8):
            idx_d[pl.ds(t * 16, 16)] = jnp.right_shift(
                b + t * 16 + lax.iota(jnp.int32, 16), 4)
        pltpu.async_copy(pr.at[idx_s], rows, sem).wait()
        pltpu.sync_copy(rows, acc.at[idx_d], add=True)
        return carry

    lax.fori_loop(0, ((PP * KNN) // 32) // 128, chunk, 0)
    plsc.subcore_barrier()
    pltpu.sync_copy(acc.at[pl.ds(rb, rows_per_tile)],
                    out.at[pl.ds(cid * PP + rb, rows_per_tile)])


def _segsum2(pooled, srcflat, zeros2):
    mesh = plsc.VectorSubcoreMesh(core_axis_name="c", subcore_axis_name="s",
                                  num_cores=2, num_subcores=16)
    k = pl.kernel(
        _segsum2_body,
        out_type=jax.ShapeDtypeStruct((2 * PP, D), jnp.float32),
        mesh=mesh,
        compiler_params=pltpu.CompilerParams(use_tc_tiling_on_sc=False, needs_layout_passes=False),
        scratch_types=[
            pltpu.VMEM((128,), jnp.int32),
            pltpu.VMEM((128,), jnp.int32),
            pltpu.VMEM((128, D), jnp.float32),
            pltpu.SemaphoreType.DMA,
            pltpu.VMEM_SHARED((PP, D), jnp.float32),
        ],
    )
    return k(pooled, srcflat, zeros2)


def _combine2_body(pool_ref, agg_ref, wl_ref, wr_ref, bf_ref, out_ref):
    a = agg_ref[...]
    mean = (a[0] + a[1]) * (1.0 / KNN)
    out_ref[...] = (jnp.dot(pool_ref[...], wl_ref[...])
                    + jnp.dot(mean, wr_ref[...]) + bf_ref[...])


def _combine2(pooled, aggf, Wlp, Wrp2, bfr):
    return pl.pallas_call(
        _combine2_body,
        grid=(PP // BLK,),
        in_specs=[
            pl.BlockSpec((BLK, D), lambda i: (i, 0)),
            pl.BlockSpec((2, BLK, D), lambda i: (0, i, 0)),
            pl.BlockSpec((D, WC), lambda i: (0, 0)),
            pl.BlockSpec((D, WC), lambda i: (0, 0)),
            pl.BlockSpec((1, WC), lambda i: (0, 0)),
        ],
        out_specs=pl.BlockSpec((BLK, WC), lambda i: (i, 0)),
        out_shape=jax.ShapeDtypeStruct((PP, WC), jnp.float32),
    )(pooled, aggf, Wlp, Wrp2, bfr)


def kernel(x, edge_index, W_vig, b_vig, W_l1, W_r1, b1, p, W_lf, W_rf, bf):
    f32 = jnp.float32
    x = x.astype(f32)
    xp = jnp.concatenate([x, jnp.zeros((NP - N, D), f32)], axis=0)
    src = edge_index[0].astype(jnp.int32)
    dst = edge_index[1].astype(jnp.int32)

    sel = jnp.zeros((D, WE), f32).at[:D, :D].set(jnp.eye(D, dtype=f32))
    e1 = jnp.zeros((1, WE), f32).at[0, D].set(1.0)
    b1r = b1.astype(f32).reshape(1, D)
    bvr = b_vig.astype(f32).reshape(1, D)
    pcol = p.astype(f32).reshape(D, 1)
    e128 = jnp.zeros((1, WE), f32).at[0, D].set(1.0)
    bcol = jnp.asarray(_BOUNDS, jnp.int32).reshape(32, 1)
    Wlp = jnp.zeros((D, WC), f32).at[:, :NCLS].set(W_lf.astype(f32))
    Wrp2 = jnp.zeros((D, WC), f32).at[:, :NCLS].set(W_rf.astype(f32))
    bfr = jnp.zeros((1, WC), f32).at[0, :NCLS].set(bf.astype(f32))
    zeros2 = jnp.zeros((PP, D), f32)

    fl, fext = _dense1(xp, W_vig.astype(f32), bvr, W_l1.astype(f32), sel, e1)

    agg2d = _fold(fext, src, dst).reshape(NP, WE)

    nk, qk = _bound(agg2d, e128, bcol)
    nkq = jnp.concatenate([nk.reshape(32), qk.reshape(32)])
    fix = _fixup(fext, src, dst, nkq).reshape(32, WE)

    hg, sc = _combine1(fl, agg2d, fix, nk.reshape(1, 32),
                       W_r1.astype(f32), b1r, pcol)

    scidx = _rank(sc, sc.reshape(1, NP))
    pooled = _poolscat(hg, scidx.reshape(NP))

    idx = _knn(pooled)

    aggf = _segsum2(pooled, idx.reshape(PP * KNN), zeros2).reshape(2, PP, D)
    outp = _combine2(pooled, aggf, Wlp, Wrp2, bfr)
    return outp[:NKEEP, :NCLS]


# speedup vs baseline: 1.8700x; 1.2772x over previous
import jax
import jax.numpy as jnp
from jax import lax
from jax.experimental import pallas as pl
from jax.experimental.pallas import tpu as pltpu
from jax.experimental.pallas import tpu_sc as plsc

N = 10000
NP = 10240
D = 128
E = 320000
KNN = 16
NKEEP = 7500
PP = 7680
WE = 144
NCLS = 10
WC = 16
BLK = 256
DUMP = 7600
RPT = NP // 32
BIGF = 1e30
BIGI = 1 << 30
_HI = jax.lax.Precision.HIGHEST

_SIZES = ([126] * 11 + [123] * 4 + [122]) * 2
_BOUNDS = []
_acc = 0
for _s in _SIZES[:-1]:
    _acc += _s * 80
    _BOUNDS.append(_acc)
_BOUNDS.append(E)


def _dense1_body(x_ref, wv_ref, bv_ref, wl_ref, sel_ref, e1_ref, fl_ref, fe_ref):
    feats = jnp.maximum(jnp.dot(x_ref[...], wv_ref[...]) + bv_ref[...], 0.0)
    fl_ref[...] = jnp.dot(feats, wl_ref[...])
    fe_ref[...] = jnp.dot(feats, sel_ref[...], precision=_HI) + e1_ref[...]


def _dense1(xp, W_vig, bvr, W_l1, sel, e1):
    return pl.pallas_call(
        _dense1_body,
        grid=(NP // BLK,),
        in_specs=[
            pl.BlockSpec((BLK, D), lambda i: (i, 0)),
            pl.BlockSpec((D, D), lambda i: (0, 0)),
            pl.BlockSpec((1, D), lambda i: (0, 0)),
            pl.BlockSpec((D, D), lambda i: (0, 0)),
            pl.BlockSpec((D, WE), lambda i: (0, 0)),
            pl.BlockSpec((1, WE), lambda i: (0, 0)),
        ],
        out_specs=[
            pl.BlockSpec((BLK, D), lambda i: (i, 0)),
            pl.BlockSpec((BLK, WE), lambda i: (i, 0)),
        ],
        out_shape=[
            jax.ShapeDtypeStruct((NP, D), jnp.float32),
            jax.ShapeDtypeStruct((NP, WE), jnp.float32),
        ],
    )(xp, W_vig, bvr, W_l1, sel, e1)


def _fold_body(fext, srcr, dstr, out, src_v, dst_v, cpak, gsrc, rows,
               sem, acc, ebuf):
    cid = lax.axis_index("c")
    sid = lax.axis_index("s")
    wid = cid * 16 + sid
    lo = wid * RPT
    hi = lo + RPT
    z16 = jnp.zeros((16,), jnp.float32)
    iota16 = lax.iota(jnp.int32, 16)

    def zero_acc(i, carry):
        acc[pl.ds(i * 16, 16)] = z16
        return carry
    lax.fori_loop(0, RPT * (WE // 16), zero_acc, 0)

    sh = E // 16
    pltpu.sync_copy(srcr.at[pl.ds(sid * sh, sh)], ebuf.at[pl.ds(sid * sh, sh)])
    pltpu.sync_copy(dstr.at[pl.ds(sid * sh, sh)],
                    ebuf.at[pl.ds(E + sid * sh, sh)])
    plsc.subcore_barrier()

    def drain(F, batch):
        iota = lax.iota(jnp.int32, 16)
        for j in range(5):
            v = cpak[pl.ds(j * 16, 16)]
            gsrc[pl.ds(j * 16, 16)] = jnp.minimum(v & 16383, N - 1)
        pltpu.async_copy(fext.at[gsrc], rows, sem).wait()

        def add_one(k, carry):
            kv = jnp.full((16,), k, jnp.int32)
            pk = plsc.load_gather(cpak, [kv])
            d = lax.shift_right_logical(pk, 14)
            for r in range(WE // 16):
                col = r * 16 + iota
                row = plsc.load_gather(rows, [kv, col])
                plsc.addupdate_scatter(acc, [d * WE + col], row)
            return carry
        lax.fori_loop(0, batch, add_one, 0)
        return F

    def full_drain(F):
        drain(F, 80)
        ts = cpak[pl.ds(80, 16)]
        cpak[pl.ds(0, 16)] = ts
        return F - 80

    def chunk(c, F):
        b = pl.multiple_of(c * 256, 8)
        pltpu.sync_copy(ebuf.at[pl.ds(b, 256)], src_v)
        pltpu.sync_copy(ebuf.at[pl.ds(E + b, 256)], dst_v)
        for g in range(16):
            vd = dst_v[pl.ds(g * 16, 16)]
            vs = src_v[pl.ds(g * 16, 16)]
            m = (vd >= lo) & (vd < hi)
            key = iota16 + jnp.where(m, 0, 64)
            pak = vs | jnp.left_shift(vd - lo, 14)
            cpak[pl.ds(F, 16)] = plsc.sort_key_val(key, pak)[1]
            F = F + jnp.max(plsc.all_reduce_population_count(m))
            F = lax.cond(F >= 80, full_drain, lambda f: f, F)
        return F

    F = lax.fori_loop(0, E // 256, chunk, jnp.int32(0))
    lax.cond(F > 0, lambda f: drain(f, f), lambda f: f, F)
    pltpu.sync_copy(acc, out.at[pl.ds(wid * (RPT * WE), RPT * WE)])


def _fold(fext, src, dst):
    mesh = plsc.VectorSubcoreMesh(core_axis_name="c", subcore_axis_name="s",
                                  num_cores=2, num_subcores=16)
    k = pl.kernel(
        _fold_body,
        out_type=jax.ShapeDtypeStruct((NP * WE,), jnp.float32),
        mesh=mesh,
        compiler_params=pltpu.CompilerParams(use_tc_tiling_on_sc=False, needs_layout_passes=False),
        scratch_types=[
            pltpu.VMEM((256,), jnp.int32),
            pltpu.VMEM((256,), jnp.int32),
            pltpu.VMEM((96,), jnp.int32),
            pltpu.VMEM((80,), jnp.int32),
            pltpu.VMEM((80, WE), jnp.float32),
            pltpu.SemaphoreType.DMA,
            pltpu.VMEM((RPT * WE,), jnp.float32),
            pltpu.VMEM_SHARED((2 * E,), jnp.int32),
        ],
    )
    return k(fext, src, dst)


def _bound_body(agg_ref, e128_ref, b_ref, nk_ref, qk_ref):
    cnt = lax.dot_general(e128_ref[...], agg_ref[...],
                          (((1,), (1,)), ((), ())), precision=_HI)
    ci = cnt
    sh = 1
    while sh < NP:
        ci = ci + jnp.concatenate(
            [jnp.zeros((1, sh), jnp.float32), ci[:, :NP - sh]], axis=1)
        sh *= 2
    cexc = ci - cnt
    b = b_ref[...].astype(jnp.float32)
    le = (cexc <= b).astype(jnp.int32)
    nk = jnp.sum(le, axis=1, keepdims=True) - 1
    ji = lax.broadcasted_iota(jnp.int32, (32, NP), 1)
    cnk = jnp.sum(jnp.where(ji == nk, cexc, 0.0), axis=1, keepdims=True)
    qk = b - cnk
    nk_ref[...] = nk
    qk_ref[...] = qk.astype(jnp.int32)


def _bound(agg2d, e128, bcol):
    return pl.pallas_call(
        _bound_body,
        grid=(1,),
        in_specs=[
            pl.BlockSpec((NP, WE), lambda i: (0, 0)),
            pl.BlockSpec((1, WE), lambda i: (0, 0)),
            pl.BlockSpec((32, 1), lambda i: (0, 0)),
        ],
        out_specs=[
            pl.BlockSpec((32, 1), lambda i: (0, 0)),
            pl.BlockSpec((32, 1), lambda i: (0, 0)),
        ],
        out_shape=[
            jax.ShapeDtypeStruct((32, 1), jnp.int32),
            jax.ShapeDtypeStruct((32, 1), jnp.int32),
        ],
    )(agg2d, e128, bcol)


def _fixup_body(fext, srcr, dstr, nkq, out, src_v, dst_v, buf, rows, rowb,
                nq_v, sem):
    cid = lax.axis_index("c")
    sid = lax.axis_index("s")
    t = cid * 16 + sid
    pltpu.sync_copy(nkq, nq_v)
    tv = jnp.full((16,), t, jnp.int32)
    n = plsc.load_gather(nq_v, [tv])
    q = plsc.load_gather(nq_v, [tv + 32])
    zi16 = jnp.zeros((16,), jnp.int32)
    for i in range(128):
        buf[pl.ds(i * 16, 16)] = zi16

    def chunk(c, F):
        b = pl.multiple_of(c * 256, 8)
        pltpu.sync_copy(dstr.at[pl.ds(b, 256)], dst_v)
        nmatch = jnp.int32(0)
        for g in range(16):
            vd = dst_v[pl.ds(g * 16, 16)]
            nmatch = nmatch + jnp.max(
                plsc.all_reduce_population_count(vd == n))

        def take(F):
            pltpu.sync_copy(srcr.at[pl.ds(b, 256)], src_v)
            iota16 = lax.iota(jnp.int32, 16)
            for g in range(16):
                vd = dst_v[pl.ds(g * 16, 16)]
                vs = src_v[pl.ds(g * 16, 16)]
                m = vd == n
                key = iota16 + jnp.where(m, 0, 64)
                buf[pl.ds(F, 16)] = plsc.sort_key_val(key, vs)[1]
                F = F + jnp.max(plsc.all_reduce_population_count(m))
            return F
        return lax.cond((nmatch > 0) & (F < 1700), take, lambda f: f, F)

    F = lax.fori_loop(0, E // 256, chunk, jnp.int32(0))

    nz = [jnp.zeros((16,), jnp.float32)] * (WE // 16)

    def fold_chunk(c, carry):
        accs, stas = carry
        pltpu.async_copy(fext.at[buf.at[pl.ds(c * 80, 80)]], rows, sem).wait()

        def fold_one(i, carry2):
            accs, stas = carry2
            egv = jnp.full((16,), c * 80 + i, jnp.int32)
            isq = egv == q
            valid = egv < F
            iv = jnp.full((16,), i, jnp.int32)
            iota = lax.iota(jnp.int32, 16)
            na, ns = [], []
            for r in range(WE // 16):
                a, s = accs[r], stas[r]
                s = jnp.where(isq, a, s)
                a = jnp.where(isq, 0.0, a)
                row = plsc.load_gather(rows, [iv, r * 16 + iota])
                a = a + jnp.where(valid, row, 0.0)
                na.append(a)
                ns.append(s)
            return (na, ns)
        return lax.fori_loop(0, 80, fold_one, (accs, stas))

    nch = (F + 79) // 80
    accs, stas = lax.fori_loop(0, nch, fold_chunk, (nz, nz))
    for r in range(WE // 16):
        rowb[pl.ds(r * 16, 16)] = stas[r] + accs[r]
    pltpu.sync_copy(rowb, out.at[pl.ds(t * WE, WE)])


def _fixup(fext, src, dst, nkq):
    mesh = plsc.VectorSubcoreMesh(core_axis_name="c", subcore_axis_name="s",
                                  num_cores=2, num_subcores=16)
    k = pl.kernel(
        _fixup_body,
        out_type=jax.ShapeDtypeStruct((32 * WE,), jnp.float32),
        mesh=mesh,
        compiler_params=pltpu.CompilerParams(use_tc_tiling_on_sc=False, needs_layout_passes=False),
        scratch_types=[
            pltpu.VMEM((256,), jnp.int32),
            pltpu.VMEM((256,), jnp.int32),
            pltpu.VMEM((2048,), jnp.int32),
            pltpu.VMEM((80, WE), jnp.float32),
            pltpu.VMEM((WE,), jnp.float32),
            pltpu.VMEM((64,), jnp.int32),
            pltpu.SemaphoreType.DMA,
        ],
    )
    return k(fext, src, dst, nkq)


def _combine1_body(fl_ref, agg_ref, fix_ref, nk_ref, wr_ref, b1_ref, p_ref,
                   hg_ref, sc_ref):
    rowids = (lax.broadcasted_iota(jnp.int32, (BLK, 1), 0)
              + pl.program_id(0) * BLK)
    m_all = (rowids == nk_ref[...]).astype(jnp.float32)
    corr = jnp.dot(m_all, fix_ref[...], precision=_HI)
    anym = jnp.sum(m_all, axis=1, keepdims=True) > 0.0
    a = jnp.where(anym, corr, agg_ref[...])
    aggf = lax.slice(a, (0, 0), (BLK, D))
    cnt = lax.slice(a, (0, D), (BLK, D + 1))
    mean = aggf / jnp.maximum(cnt, 1.0)
    h = fl_ref[...] + jnp.dot(mean, wr_ref[...]) + b1_ref[...]
    pcol = p_ref[...]
    s = jnp.dot(h, pcol) / jnp.sqrt(jnp.sum(pcol * pcol))
    s = jnp.where(rowids < N, s, -BIGF)
    sc_ref[...] = s
    hg_ref[...] = h * jnp.tanh(s)


def _combine1(fl, agg2d, fix, nkrow, W_r1, b1r, pcol):
    return pl.pallas_call(
        _combine1_body,
        grid=(NP // BLK,),
        in_specs=[
            pl.BlockSpec((BLK, D), lambda i: (i, 0)),
            pl.BlockSpec((BLK, WE), lambda i: (i, 0)),
            pl.BlockSpec((32, WE), lambda i: (0, 0)),
            pl.BlockSpec((1, 32), lambda i: (0, 0)),
            pl.BlockSpec((D, D), lambda i: (0, 0)),
            pl.BlockSpec((1, D), lambda i: (0, 0)),
            pl.BlockSpec((D, 1), lambda i: (0, 0)),
        ],
        out_specs=[
            pl.BlockSpec((BLK, D), lambda i: (i, 0)),
            pl.BlockSpec((BLK, 1), lambda i: (i, 0)),
        ],
        out_shape=[
            jax.ShapeDtypeStruct((NP, D), jnp.float32),
            jax.ShapeDtypeStruct((NP, 1), jnp.float32),
        ],
    )(fl, agg2d, fix, nkrow, W_r1, b1r, pcol)


def _rank_body(scol_ref, srow_ref, out_ref):
    si = scol_ref[...]
    sj = srow_ref[...]
    gt = sj > si
    eq = sj == si
    ji = lax.broadcasted_iota(jnp.int32, (BLK, NP), 1)
    ii = (lax.broadcasted_iota(jnp.int32, (BLK, NP), 0)
          + pl.program_id(0) * BLK)
    cond = gt | (eq & (ji < ii))
    rank = jnp.sum(cond.astype(jnp.int32), axis=1, keepdims=True)
    out_ref[...] = jnp.where(rank < NKEEP, rank, DUMP)


def _rank(scol, srow):
    return pl.pallas_call(
        _rank_body,
        grid=(NP // BLK,),
        in_specs=[
            pl.BlockSpec((BLK, 1), lambda i: (i, 0)),
            pl.BlockSpec((1, NP), lambda i: (0, 0)),
        ],
        out_specs=pl.BlockSpec((BLK, 1), lambda i: (i, 0)),
        out_shape=jax.ShapeDtypeStruct((NP, 1), jnp.int32),
    )(scol, srow)


def _poolscat_body(hg, sidx, out, idx_v, rows, sem):
    cid = lax.axis_index("c")
    sid = lax.axis_index("s")
    wid = cid * 16 + sid
    base = wid * (NP // 32)
    for c in range((NP // 32) // 64):
        b = pl.multiple_of(base + c * 64, 8)
        pltpu.sync_copy(sidx.at[pl.ds(b, 64)], idx_v)
        pltpu.sync_copy(hg.at[pl.ds(b, 64)], rows)
        pltpu.async_copy(rows, out.at[idx_v], sem).wait()


def _poolscat(hg, sidx):
    mesh = plsc.VectorSubcoreMesh(core_axis_name="c", subcore_axis_name="s",
                                  num_cores=2, num_subcores=16)
    k = pl.kernel(
        _poolscat_body,
        out_type=jax.ShapeDtypeStruct((PP, D), jnp.float32),
        mesh=mesh,
        compiler_params=pltpu.CompilerParams(use_tc_tiling_on_sc=False, needs_layout_passes=False),
        scratch_types=[
            pltpu.VMEM((64,), jnp.int32),
            pltpu.VMEM((64, D), jnp.float32),
            pltpu.SemaphoreType.DMA,
        ],
    )
    return k(hg, sidx)


def _knn_body(q_ref, p_ref, idx_ref):
    q = q_ref[...]
    pall = p_ref[...]
    g = lax.dot_general(q, pall, (((1,), (1,)), ((), ())))
    ones = jnp.ones((1, D), jnp.float32)
    sq_row = lax.dot_general(ones, pall * pall, (((1,), (1,)), ((), ())),
                             precision=_HI)
    sq_col = jnp.sum(q * q, axis=1, keepdims=True)
    cur = (sq_col + sq_row) - 2.0 * g
    jcol = lax.broadcasted_iota(jnp.int32, (BLK, PP), 1)
    cur = jnp.where(jcol >= NKEEP, BIGF, cur)
    cols = []
    for _ in range(KNN):
        m = jnp.min(cur, axis=1, keepdims=True)
        cand = jnp.where(cur == m, jcol, BIGI)
        sel = jnp.min(cand, axis=1, keepdims=True)
        sel = jnp.minimum(sel, PP - 1)
        cols.append(sel)
        cur = jnp.where(jcol == sel, BIGF, cur)
    idx_ref[...] = jnp.concatenate(cols, axis=1)


def _knn(pooled):
    return pl.pallas_call(
        _knn_body,
        grid=(PP // BLK,),
        in_specs=[
            pl.BlockSpec((BLK, D), lambda i: (i, 0)),
            pl.BlockSpec((PP, D), lambda i: (0, 0)),
        ],
        out_specs=pl.BlockSpec((BLK, KNN), lambda i: (i, 0)),
        out_shape=jax.ShapeDtypeStruct((PP, KNN), jnp.int32),
    )(pooled, pooled)


def _segsum2_body(pr, srcr, zz, out, idx_s, idx_d, rows, sem, acc):
    cid = lax.axis_index("c")
    sid = lax.axis_index("s")
    rows_per_tile = PP // 16
    rb = sid * rows_per_tile
    pltpu.sync_copy(zz.at[pl.ds(rb, rows_per_tile)],
                    acc.at[pl.ds(rb, rows_per_tile)])
    plsc.subcore_barrier()
    wid = cid * 16 + sid
    ebase = wid * ((PP * KNN) // 32)

    def chunk(c, carry):
        b = pl.multiple_of(ebase + c * 128, 8)
        pltpu.sync_copy(srcr.at[pl.ds(b, 128)], idx_s)
        for t in range(8):
            idx_d[pl.ds(t * 16, 16)] = jnp.right_shift(
                b + t * 16 + lax.iota(jnp.int32, 16), 4)
        pltpu.async_copy(pr.at[idx_s], rows, sem).wait()
        pltpu.sync_copy(rows, acc.at[idx_d], add=True)
        return carry

    lax.fori_loop(0, ((PP * KNN) // 32) // 128, chunk, 0)
    plsc.subcore_barrier()
    pltpu.sync_copy(acc.at[pl.ds(rb, rows_per_tile)],
                    out.at[pl.ds(cid * PP + rb, rows_per_tile)])


def _segsum2(pooled, srcflat, zeros2):
    mesh = plsc.VectorSubcoreMesh(core_axis_name="c", subcore_axis_name="s",
                                  num_cores=2, num_subcores=16)
    k = pl.kernel(
        _segsum2_body,
        out_type=jax.ShapeDtypeStruct((2 * PP, D), jnp.float32),
        mesh=mesh,
        compiler_params=pltpu.CompilerParams(use_tc_tiling_on_sc=False, needs_layout_passes=False),
        scratch_types=[
            pltpu.VMEM((128,), jnp.int32),
            pltpu.VMEM((128,), jnp.int32),
            pltpu.VMEM((128, D), jnp.float32),
            pltpu.SemaphoreType.DMA,
            pltpu.VMEM_SHARED((PP, D), jnp.float32),
        ],
    )
    return k(pooled, srcflat, zeros2)


def _combine2_body(pool_ref, agg_ref, wl_ref, wr_ref, bf_ref, out_ref):
    a = agg_ref[...]
    mean = (a[0] + a[1]) * (1.0 / KNN)
    out_ref[...] = (jnp.dot(pool_ref[...], wl_ref[...])
                    + jnp.dot(mean, wr_ref[...]) + bf_ref[...])


def _combine2(pooled, aggf, Wlp, Wrp2, bfr):
    return pl.pallas_call(
        _combine2_body,
        grid=(PP // BLK,),
        in_specs=[
            pl.BlockSpec((BLK, D), lambda i: (i, 0)),
            pl.BlockSpec((2, BLK, D), lambda i: (0, i, 0)),
            pl.BlockSpec((D, WC), lambda i: (0, 0)),
            pl.BlockSpec((D, WC), lambda i: (0, 0)),
            pl.BlockSpec((1, WC), lambda i: (0, 0)),
        ],
        out_specs=pl.BlockSpec((BLK, WC), lambda i: (i, 0)),
        out_shape=jax.ShapeDtypeStruct((PP, WC), jnp.float32),
    )(pooled, aggf, Wlp, Wrp2, bfr)


def kernel(x, edge_index, W_vig, b_vig, W_l1, W_r1, b1, p, W_lf, W_rf, bf):
    f32 = jnp.float32
    x = x.astype(f32)
    xp = jnp.concatenate([x, jnp.zeros((NP - N, D), f32)], axis=0)
    src = edge_index[0].astype(jnp.int32)
    dst = edge_index[1].astype(jnp.int32)

    sel = jnp.zeros((D, WE), f32).at[:D, :D].set(jnp.eye(D, dtype=f32))
    e1 = jnp.zeros((1, WE), f32).at[0, D].set(1.0)
    b1r = b1.astype(f32).reshape(1, D)
    bvr = b_vig.astype(f32).reshape(1, D)
    pcol = p.astype(f32).reshape(D, 1)
    e128 = jnp.zeros((1, WE), f32).at[0, D].set(1.0)
    bcol = jnp.asarray(_BOUNDS, jnp.int32).reshape(32, 1)
    Wlp = jnp.zeros((D, WC), f32).at[:, :NCLS].set(W_lf.astype(f32))
    Wrp2 = jnp.zeros((D, WC), f32).at[:, :NCLS].set(W_rf.astype(f32))
    bfr = jnp.zeros((1, WC), f32).at[0, :NCLS].set(bf.astype(f32))
    zeros2 = jnp.zeros((PP, D), f32)

    fl, fext = _dense1(xp, W_vig.astype(f32), bvr, W_l1.astype(f32), sel, e1)

    agg2d = _fold(fext, src, dst).reshape(NP, WE)

    nk, qk = _bound(agg2d, e128, bcol)
    nkq = jnp.concatenate([nk.reshape(32), qk.reshape(32)])
    fix = _fixup(fext, src, dst, nkq).reshape(32, WE)

    hg, sc = _combine1(fl, agg2d, fix, nk.reshape(1, 32),
                       W_r1.astype(f32), b1r, pcol)

    scidx = _rank(sc, sc.reshape(1, NP))
    pooled = _poolscat(hg, scidx.reshape(NP))

    idx = _knn(pooled)

    aggf = _segsum2(pooled, idx.reshape(PP * KNN), zeros2).reshape(2, PP, D)
    outp = _combine2(pooled, aggf, Wlp, Wrp2, bfr)
    return outp[:NKEEP, :NCLS]
